# baseline (speedup 1.0000x reference)
import functools
import jax
import jax.numpy as jnp
from jax.experimental import pallas as pl

K = 20


def _knn_idx(x, k):
    xx = jnp.sum(x * x, axis=1)
    inner = jnp.einsum('bcn,bcm->bnm', x, x)
    neg_d = -xx[:, :, None] + 2.0 * inner - xx[:, None, :]
    _, idx = jax.lax.top_k(neg_d, k)
    return idx


def _gather_feats(feats_t, idx):
    return jax.vmap(lambda f, i: f[i])(feats_t, idx)


def _get_graph_feature(x, k):
    idx = _knn_idx(x, k)
    xt = jnp.transpose(x, (0, 2, 1))
    feat = _gather_feats(xt, idx)
    center = jnp.broadcast_to(xt[:, :, None, :], feat.shape)
    edge = jnp.concatenate([feat - center, center], axis=-1)
    return jnp.transpose(edge, (0, 3, 1, 2))


def _edge_conv(x, layers, k):
    h = _get_graph_feature(x, k)
    for (w, g, b) in layers:
        h = jnp.einsum('oc,bcnk->bonk', w, h)
        h = h * g[None, :, None, None] + b[None, :, None, None]
        h = jax.nn.leaky_relu(h, 0.2)
    return jnp.max(h, axis=-1)


def _mlp(x, p):
    w, g, b = p
    h = jnp.einsum('oc,bcn->bon', w, x)
    h = h * g[None, :, None] + b[None, :, None]
    return jax.nn.leaky_relu(h, 0.2)


def _rand_pool(node, feats, m, k):
    B, _, N = node.shape
    stride = N // m
    sel = jnp.arange(m) * stride
    new_node = node[:, :, sel]
    xx_new = jnp.sum(new_node * new_node, axis=1)
    xx_old = jnp.sum(node * node, axis=1)
    inner = jnp.einsum('bcm,bcn->bmn', new_node, node)
    neg_d = -xx_new[:, :, None] + 2.0 * inner - xx_old[:, None, :]
    _, idx = jax.lax.top_k(neg_d, k)
    ft = jnp.transpose(feats, (0, 2, 1))
    grouped = _gather_feats(ft, idx)
    new_feats = jnp.transpose(jnp.max(grouped, axis=2), (0, 2, 1))
    return new_node, new_feats


def _unpool(node_src, node_dst, x):
    xx_d = jnp.sum(node_dst * node_dst, axis=1)
    xx_s = jnp.sum(node_src * node_src, axis=1)
    inner = jnp.einsum('bcn,bcm->bnm', node_dst, node_src)
    neg_d = -xx_d[:, :, None] + 2.0 * inner - xx_s[:, None, :]
    idx = jnp.argmax(neg_d, axis=-1)
    xt = jnp.transpose(x, (0, 2, 1))
    out = jax.vmap(lambda f, i: f[i])(xt, idx)
    return jnp.transpose(out, (0, 2, 1))


def _final_conv_body(w_ref, h_ref, o_ref):
    o_ref[...] = jnp.dot(h_ref[...], w_ref[...].T,
                         preferred_element_type=jnp.float32)


def _final_conv(w, h):
    B, C, N = h.shape
    O = w.shape[0]
    ht = jnp.transpose(h, (0, 2, 1))
    out = pl.pallas_call(
        _final_conv_body,
        grid=(B,),
        in_specs=[
            pl.BlockSpec((O, C), lambda b: (0, 0)),
            pl.BlockSpec((1, N, C), lambda b: (b, 0, 0)),
        ],
        out_specs=pl.BlockSpec((1, N, O), lambda b: (b, 0, 0)),
        out_shape=jax.ShapeDtypeStruct((B, N, O), jnp.float32),
    )(w, ht)
    return jnp.transpose(out, (0, 2, 1))


def kernel(x, params):
    N = x.shape[-1]
    node0 = x[:, :3, :]
    h = _edge_conv(x, params['ec1'], K)
    x0 = _edge_conv(h, params['ec2'], K)
    x_t0 = jnp.max(_mlp(x0, params['pn3']), axis=-1, keepdims=True)
    node1, node1_feats = _rand_pool(node0, x0, N // 4, K)
    h = _edge_conv(node1_feats, params['ec4'], K)
    x1 = _edge_conv(h, params['ec5'], K)
    x_t1 = jnp.max(_mlp(x1, params['pn6']), axis=-1, keepdims=True)
    x1 = jax.nn.relu(x1 + node1_feats)
    node2, node2_feats = _rand_pool(node1, x1, N // 16, K)
    h = _edge_conv(node2_feats, params['ec7'], K)
    x2 = _edge_conv(h, params['ec8'], K)
    x_t2 = jnp.max(_mlp(x2, params['pn9']), axis=-1, keepdims=True)
    x2 = jax.nn.relu(x2 + node2_feats)
    node3, node3_feats = _rand_pool(node2, x2, N // 64, K)
    h = _edge_conv(node3_feats, params['ec10'], K // 2)
    x3 = _edge_conv(h, params['ec11'], K // 2)
    x_t3 = jnp.max(_mlp(x3, params['pn12']), axis=-1, keepdims=True)
    x3 = jax.nn.relu(x3 + node3_feats)
    g = jnp.concatenate([x_t0, x_t1, x_t2, x_t3], axis=-1)
    g = jnp.max(g, axis=-1, keepdims=True)
    g = jnp.tile(g, (1, 1, x3.shape[-1]))
    h = jnp.concatenate([g, x3], axis=1)
    h = _mlp(h, params['pn13'])
    h = _unpool(node3, node2, h)
    h = jnp.concatenate([h, x2], axis=1)
    h = _mlp(h, params['pn14'])
    h = _unpool(node2, node1, h)
    h = jnp.concatenate([h, x1], axis=1)
    h = _mlp(h, params['pn15'])
    h = _unpool(node1, node0, h)
    h = jnp.concatenate([h, x0], axis=1)
    h = _mlp(h, params['pn16'])
    out = _final_conv(params['conv17'], h)
    return out, node1, node2, node3


# speedup vs baseline: 1.7358x; 1.7358x over previous
import functools
import jax
import jax.numpy as jnp
from jax.experimental import pallas as pl

K = 20
NEG = float('-inf')
F32 = jnp.float32


def _lrelu(x):
    return jnp.where(x >= 0, x, 0.2 * x)


def _blk(m, cap=256):
    b = min(cap, m)
    while m % b:
        b -= 1
    return b


def _knn_body(k, n, q_ref, t_ref, o_ref):
    b = pl.program_id(0)
    q = q_ref[0]
    t = t_ref[0]
    d = 2.0 * jax.lax.dot_general(q, t, (((1,), (1,)), ((), ())),
                                  preferred_element_type=F32)
    d = d - jnp.sum(q * q, axis=1, keepdims=True)
    d = d - jnp.sum(t * t, axis=1)[None, :]
    iota = jax.lax.broadcasted_iota(jnp.int32, d.shape, 1)
    cols = []
    for _ in range(k):
        m = jnp.max(d, axis=-1, keepdims=True)
        sel = jnp.min(jnp.where(d == m, iota, n), axis=-1, keepdims=True)
        cols.append(sel)
        if k > 1:
            d = jnp.where(iota == sel, NEG, d)
    idx = cols[0] if k == 1 else jnp.concatenate(cols, axis=1)
    o_ref[0] = idx + b * n


def _knn(qt, tt, k):
    B, M, C = qt.shape
    N = tt.shape[1]
    blk = _blk(M)
    return pl.pallas_call(
        functools.partial(_knn_body, k, N),
        grid=(B, M // blk),
        in_specs=[
            pl.BlockSpec((1, blk, C), lambda b, i: (b, i, 0)),
            pl.BlockSpec((1, N, C), lambda b, i: (b, 0, 0)),
        ],
        out_specs=pl.BlockSpec((1, blk, k), lambda b, i: (b, i, 0)),
        out_shape=jax.ShapeDtypeStruct((B, M, k), jnp.int32),
    )(qt, tt)


def _gather_body(r, i_ref, t_ref, o_ref):
    idx = i_ref[0, 0]
    iota = jax.lax.broadcasted_iota(jnp.int32, (idx.shape[0], r), 1)
    onehot = (idx[:, None] == iota).astype(F32)
    o_ref[...] = jnp.dot(onehot, t_ref[...], preferred_element_type=F32,
                         precision=jax.lax.Precision.HIGHEST)


def _gather(table, idx_flat):
    R, D = table.shape
    Mf = idx_flat.shape[0]
    blk = _blk(Mf)
    idx3 = idx_flat.reshape(Mf // blk, 1, blk)
    return pl.pallas_call(
        functools.partial(_gather_body, R),
        grid=(Mf // blk,),
        in_specs=[
            pl.BlockSpec((1, 1, blk), lambda i: (i, 0, 0)),
            pl.BlockSpec((R, D), lambda i: (0, 0)),
        ],
        out_specs=pl.BlockSpec((blk, D), lambda i: (i, 0)),
        out_shape=jax.ShapeDtypeStruct((Mf, D), F32),
    )(idx3, table)


def _mm_body(act, x_ref, w_ref, gb_ref, o_ref):
    y = jnp.dot(x_ref[...], w_ref[...], preferred_element_type=F32)
    y = y * gb_ref[0][None, :] + gb_ref[1][None, :]
    o_ref[...] = _lrelu(y) if act else y


def _mm_plain_body(x_ref, w_ref, o_ref):
    o_ref[...] = jnp.dot(x_ref[...], w_ref[...], preferred_element_type=F32)


def _mm(xf, wt, gb=None, act=False):
    Mf, Cin = xf.shape
    O = wt.shape[1]
    blk = _blk(Mf)
    if gb is None:
        return pl.pallas_call(
            _mm_plain_body,
            grid=(Mf // blk,),
            in_specs=[
                pl.BlockSpec((blk, Cin), lambda i: (i, 0)),
                pl.BlockSpec((Cin, O), lambda i: (0, 0)),
            ],
            out_specs=pl.BlockSpec((blk, O), lambda i: (i, 0)),
            out_shape=jax.ShapeDtypeStruct((Mf, O), F32),
        )(xf, wt)
    return pl.pallas_call(
        functools.partial(_mm_body, act),
        grid=(Mf // blk,),
        in_specs=[
            pl.BlockSpec((blk, Cin), lambda i: (i, 0)),
            pl.BlockSpec((Cin, O), lambda i: (0, 0)),
            pl.BlockSpec((2, O), lambda i: (0, 0)),
        ],
        out_specs=pl.BlockSpec((blk, O), lambda i: (i, 0)),
        out_shape=jax.ShapeDtypeStruct((Mf, O), F32),
    )(xf, wt, gb)


def _mm_max_body(x_ref, w_ref, gb_ref, o_ref):
    i = pl.program_id(1)
    y = jnp.dot(x_ref[0], w_ref[...], preferred_element_type=F32)
    y = _lrelu(y * gb_ref[0][None, :] + gb_ref[1][None, :])
    blkmax = jnp.max(y, axis=0, keepdims=True)

    @pl.when(i == 0)
    def _():
        o_ref[0] = blkmax

    @pl.when(i > 0)
    def _():
        o_ref[0] = jnp.maximum(o_ref[0], blkmax)


def _mm_max(xt, wt, gb):
    B, M, Cin = xt.shape
    O = wt.shape[1]
    blk = _blk(M)
    return pl.pallas_call(
        _mm_max_body,
        grid=(B, M // blk),
        in_specs=[
            pl.BlockSpec((1, blk, Cin), lambda b, i: (b, i, 0)),
            pl.BlockSpec((Cin, O), lambda b, i: (0, 0)),
            pl.BlockSpec((2, O), lambda b, i: (0, 0)),
        ],
        out_specs=pl.BlockSpec((1, 1, O), lambda b, i: (b, 0, 0)),
        out_shape=jax.ShapeDtypeStruct((B, 1, O), F32),
    )(xt, wt, gb)


def _edgec_body(k, nlayers, has_res, fg_ref, fi_ref, *refs):
    ws = refs[:2 * nlayers]
    rest = refs[2 * nlayers:]
    if has_res:
        res_ref, pre_ref, post_ref = rest
    else:
        (pre_ref,) = rest
    fi = fi_ref[...]
    acc = None
    for j in range(k):
        e = jnp.concatenate([fg_ref[:, j, :] - fi, fi], axis=1)
        h = e
        for li in range(nlayers):
            w_ref, gb_ref = ws[2 * li], ws[2 * li + 1]
            h = jnp.dot(h, w_ref[...], preferred_element_type=F32)
            h = _lrelu(h * gb_ref[0][None, :] + gb_ref[1][None, :])
        acc = h if acc is None else jnp.maximum(acc, h)
    pre_ref[...] = acc
    if has_res:
        post_ref[...] = jnp.maximum(acc + res_ref[...], 0.0)


def _ec(feat, k, layers, res=None):
    B, M, C = feat.shape
    ff = feat.reshape(B * M, C)
    idx = _knn(feat, feat, k)
    fg = _gather(ff, idx.reshape(-1)).reshape(B * M, k, C)
    blk = _blk(B * M)
    in_specs = [
        pl.BlockSpec((blk, k, C), lambda i: (i, 0, 0)),
        pl.BlockSpec((blk, C), lambda i: (i, 0)),
    ]
    args = [fg, ff]
    O = None
    for (w, g, b) in layers:
        o, ci = w.shape
        in_specs.append(pl.BlockSpec((ci, o), lambda i: (0, 0)))
        args.append(w.T)
        in_specs.append(pl.BlockSpec((2, o), lambda i: (0, 0)))
        args.append(jnp.stack([g, b]))
        O = o
    shp = jax.ShapeDtypeStruct((B * M, O), F32)
    if res is None:
        return pl.pallas_call(
            functools.partial(_edgec_body, k, len(layers), False),
            grid=(B * M // blk,),
            in_specs=in_specs,
            out_specs=pl.BlockSpec((blk, O), lambda i: (i, 0)),
            out_shape=shp,
        )(*args)
    in_specs.append(pl.BlockSpec((blk, O), lambda i: (i, 0)))
    args.append(res)
    return pl.pallas_call(
        functools.partial(_edgec_body, k, len(layers), True),
        grid=(B * M // blk,),
        in_specs=in_specs,
        out_specs=[pl.BlockSpec((blk, O), lambda i: (i, 0))] * 2,
        out_shape=[shp, shp],
    )(*args)


def _pool_body(k, g_ref, o_ref):
    acc = g_ref[:, 0, :]
    for j in range(1, k):
        acc = jnp.maximum(acc, g_ref[:, j, :])
    o_ref[...] = acc


def _pool(g3):
    Mq, k, C = g3.shape
    blk = _blk(Mq)
    return pl.pallas_call(
        functools.partial(_pool_body, k),
        grid=(Mq // blk,),
        in_specs=[pl.BlockSpec((blk, k, C), lambda i: (i, 0, 0))],
        out_specs=pl.BlockSpec((blk, C), lambda i: (i, 0)),
        out_shape=jax.ShapeDtypeStruct((Mq, C), F32),
    )(g3)


def _gb(p):
    w, g, b = p
    return w.T, jnp.stack([g, b])


def kernel(x, params):
    B, _, N = x.shape
    xt = jnp.transpose(x, (0, 2, 1))
    node0t = xt[..., :3]
    node1t = node0t[:, ::4, :]
    node2t = node1t[:, ::4, :]
    node3t = node2t[:, ::4, :]
    M1, M2, M3 = N // 4, N // 16, N // 64

    h = _ec(xt, K, params['ec1'])
    x0 = _ec(h.reshape(B, N, 64), K, params['ec2'])
    xt0 = _mm_max(x0.reshape(B, N, 64), *_gb(params['pn3']))

    idxp = _knn(node1t, node0t, K)
    n1f = _pool(_gather(x0, idxp.reshape(-1)).reshape(B * M1, K, 64))

    h = _ec(n1f.reshape(B, M1, 64), K, params['ec4'])
    x1pre, x1 = _ec(h.reshape(B, M1, 64), K, params['ec5'], res=n1f)
    xt1 = _mm_max(x1pre.reshape(B, M1, 64), *_gb(params['pn6']))

    idxp = _knn(node2t, node1t, K)
    n2f = _pool(_gather(x1, idxp.reshape(-1)).reshape(B * M2, K, 64))

    h = _ec(n2f.reshape(B, M2, 64), K, params['ec7'])
    x2pre, x2 = _ec(h.reshape(B, M2, 64), K, params['ec8'], res=n2f)
    xt2 = _mm_max(x2pre.reshape(B, M2, 64), *_gb(params['pn9']))

    idxp = _knn(node3t, node2t, K)
    n3f = _pool(_gather(x2, idxp.reshape(-1)).reshape(B * M3, K, 64))

    h = _ec(n3f.reshape(B, M3, 64), K // 2, params['ec10'])
    x3pre, x3 = _ec(h.reshape(B, M3, 64), K // 2, params['ec11'], res=n3f)
    xt3 = _mm_max(x3pre.reshape(B, M3, 64), *_gb(params['pn12']))

    g = jnp.maximum(jnp.maximum(xt0, xt1), jnp.maximum(xt2, xt3))
    gt = jnp.broadcast_to(g, (B, M3, 1024))
    cat = jnp.concatenate([gt, x3.reshape(B, M3, 64)], axis=-1)
    h = _mm(cat.reshape(B * M3, 1088), *_gb(params['pn13']), act=True)

    idxu = _knn(node2t, node3t, 1)
    hu = _gather(h, idxu.reshape(-1))
    cat = jnp.concatenate([hu, x2], axis=-1)
    h = _mm(cat, *_gb(params['pn14']), act=True)

    idxu = _knn(node1t, node2t, 1)
    hu = _gather(h, idxu.reshape(-1))
    cat = jnp.concatenate([hu, x1], axis=-1)
    h = _mm(cat, *_gb(params['pn15']), act=True)

    idxu = _knn(node0t, node1t, 1)
    hu = _gather(h, idxu.reshape(-1))
    cat = jnp.concatenate([hu, x0], axis=-1)
    h = _mm(cat, *_gb(params['pn16']), act=True)

    out = _mm(h, params['conv17'].T)
    out = jnp.transpose(out.reshape(B, N, 13), (0, 2, 1))
    return (out,
            jnp.transpose(node1t, (0, 2, 1)),
            jnp.transpose(node2t, (0, 2, 1)),
            jnp.transpose(node3t, (0, 2, 1)))


# speedup vs baseline: 2.7654x; 1.5932x over previous
import functools
import jax
import jax.numpy as jnp
from jax.experimental import pallas as pl

K = 20
NEG = float('-inf')
F32 = jnp.float32


def _lrelu(x):
    return jnp.where(x >= 0, x, 0.2 * x)


def _blk(m, cap=256):
    b = min(cap, m)
    while m % b:
        b -= 1
    return b


def _knn_body(k, n, q_ref, t_ref, o_ref):
    b = pl.program_id(0)
    q = q_ref[0]
    t = t_ref[0]
    d = 2.0 * jax.lax.dot_general(q, t, (((1,), (1,)), ((), ())),
                                  preferred_element_type=F32)
    d = d - jnp.sum(q * q, axis=1, keepdims=True)
    d = d - jnp.sum(t * t, axis=1)[None, :]
    iota = jax.lax.broadcasted_iota(jnp.int32, d.shape, 1)
    cols = []
    for _ in range(k):
        m = jnp.max(d, axis=-1, keepdims=True)
        sel = jnp.min(jnp.where(d == m, iota, n), axis=-1, keepdims=True)
        cols.append(sel)
        if k > 1:
            d = jnp.where(iota == sel, NEG, d)
    idx = cols[0] if k == 1 else jnp.concatenate(cols, axis=1)
    o_ref[0] = idx + b * n


def _knn(qt, tt, k):
    B, M, C = qt.shape
    N = tt.shape[1]
    blk = _blk(M)
    return pl.pallas_call(
        functools.partial(_knn_body, k, N),
        grid=(B, M // blk),
        in_specs=[
            pl.BlockSpec((1, blk, C), lambda b, i: (b, i, 0)),
            pl.BlockSpec((1, N, C), lambda b, i: (b, 0, 0)),
        ],
        out_specs=pl.BlockSpec((1, blk, k), lambda b, i: (b, i, 0)),
        out_shape=jax.ShapeDtypeStruct((B, M, k), jnp.int32),
    )(qt, tt)


def _gather_body(r, i_ref, t_ref, o_ref):
    idx = i_ref[0, 0]
    iota = jax.lax.broadcasted_iota(jnp.int32, (idx.shape[0], r), 1)
    onehot = (idx[:, None] == iota).astype(jnp.bfloat16)
    t = t_ref[...]
    t1 = t.astype(jnp.bfloat16)
    r1 = t - t1.astype(F32)
    t2 = r1.astype(jnp.bfloat16)
    t3 = (r1 - t2.astype(F32)).astype(jnp.bfloat16)
    acc = jnp.dot(onehot, t1, preferred_element_type=F32)
    acc = acc + jnp.dot(onehot, t2, preferred_element_type=F32)
    acc = acc + jnp.dot(onehot, t3, preferred_element_type=F32)
    o_ref[...] = acc


def _gather(table, idx_flat):
    R, D = table.shape
    Mf = idx_flat.shape[0]
    blk = _blk(Mf)
    idx3 = idx_flat.reshape(Mf // blk, 1, blk)
    return pl.pallas_call(
        functools.partial(_gather_body, R),
        grid=(Mf // blk,),
        in_specs=[
            pl.BlockSpec((1, 1, blk), lambda i: (i, 0, 0)),
            pl.BlockSpec((R, D), lambda i: (0, 0)),
        ],
        out_specs=pl.BlockSpec((blk, D), lambda i: (i, 0)),
        out_shape=jax.ShapeDtypeStruct((Mf, D), F32),
    )(idx3, table)


def _mm_body(act, x_ref, w_ref, gb_ref, o_ref):
    y = jnp.dot(x_ref[...], w_ref[...], preferred_element_type=F32)
    y = y * gb_ref[0][None, :] + gb_ref[1][None, :]
    o_ref[...] = _lrelu(y) if act else y


def _mm_plain_body(x_ref, w_ref, o_ref):
    o_ref[...] = jnp.dot(x_ref[...], w_ref[...], preferred_element_type=F32)


def _mm(xf, wt, gb=None, act=False):
    Mf, Cin = xf.shape
    O = wt.shape[1]
    blk = _blk(Mf)
    if gb is None:
        return pl.pallas_call(
            _mm_plain_body,
            grid=(Mf // blk,),
            in_specs=[
                pl.BlockSpec((blk, Cin), lambda i: (i, 0)),
                pl.BlockSpec((Cin, O), lambda i: (0, 0)),
            ],
            out_specs=pl.BlockSpec((blk, O), lambda i: (i, 0)),
            out_shape=jax.ShapeDtypeStruct((Mf, O), F32),
        )(xf, wt)
    return pl.pallas_call(
        functools.partial(_mm_body, act),
        grid=(Mf // blk,),
        in_specs=[
            pl.BlockSpec((blk, Cin), lambda i: (i, 0)),
            pl.BlockSpec((Cin, O), lambda i: (0, 0)),
            pl.BlockSpec((2, O), lambda i: (0, 0)),
        ],
        out_specs=pl.BlockSpec((blk, O), lambda i: (i, 0)),
        out_shape=jax.ShapeDtypeStruct((Mf, O), F32),
    )(xf, wt, gb)


def _mm_max_body(x_ref, w_ref, gb_ref, o_ref):
    i = pl.program_id(1)
    y = jnp.dot(x_ref[0], w_ref[...], preferred_element_type=F32)
    y = _lrelu(y * gb_ref[0][None, :] + gb_ref[1][None, :])
    blkmax = jnp.max(y, axis=0, keepdims=True)

    @pl.when(i == 0)
    def _():
        o_ref[0] = blkmax

    @pl.when(i > 0)
    def _():
        o_ref[0] = jnp.maximum(o_ref[0], blkmax)


def _mm_max(xt, wt, gb):
    B, M, Cin = xt.shape
    O = wt.shape[1]
    blk = _blk(M)
    return pl.pallas_call(
        _mm_max_body,
        grid=(B, M // blk),
        in_specs=[
            pl.BlockSpec((1, blk, Cin), lambda b, i: (b, i, 0)),
            pl.BlockSpec((Cin, O), lambda b, i: (0, 0)),
            pl.BlockSpec((2, O), lambda b, i: (0, 0)),
        ],
        out_specs=pl.BlockSpec((1, 1, O), lambda b, i: (b, 0, 0)),
        out_shape=jax.ShapeDtypeStruct((B, 1, O), F32),
    )(xt, wt, gb)


def _edgec_body(k, nlayers, has_res, fg_ref, fi_ref, *refs):
    ws = refs[:2 * nlayers]
    rest = refs[2 * nlayers:]
    if has_res:
        res_ref, pre_ref, post_ref = rest
    else:
        (pre_ref,) = rest
    fi = fi_ref[...]
    acc = None
    for j in range(k):
        e = jnp.concatenate([fg_ref[:, j, :] - fi, fi], axis=1)
        h = e
        for li in range(nlayers):
            w_ref, gb_ref = ws[2 * li], ws[2 * li + 1]
            h = jnp.dot(h, w_ref[...], preferred_element_type=F32)
            h = _lrelu(h * gb_ref[0][None, :] + gb_ref[1][None, :])
        acc = h if acc is None else jnp.maximum(acc, h)
    pre_ref[...] = acc
    if has_res:
        post_ref[...] = jnp.maximum(acc + res_ref[...], 0.0)


def _ec(feat, k, layers, res=None):
    B, M, C = feat.shape
    ff = feat.reshape(B * M, C)
    idx = _knn(feat, feat, k)
    fg = _gather(ff, idx.reshape(-1)).reshape(B * M, k, C)
    blk = _blk(B * M)
    in_specs = [
        pl.BlockSpec((blk, k, C), lambda i: (i, 0, 0)),
        pl.BlockSpec((blk, C), lambda i: (i, 0)),
    ]
    args = [fg, ff]
    O = None
    for (w, g, b) in layers:
        o, ci = w.shape
        in_specs.append(pl.BlockSpec((ci, o), lambda i: (0, 0)))
        args.append(w.T)
        in_specs.append(pl.BlockSpec((2, o), lambda i: (0, 0)))
        args.append(jnp.stack([g, b]))
        O = o
    shp = jax.ShapeDtypeStruct((B * M, O), F32)
    if res is None:
        return pl.pallas_call(
            functools.partial(_edgec_body, k, len(layers), False),
            grid=(B * M // blk,),
            in_specs=in_specs,
            out_specs=pl.BlockSpec((blk, O), lambda i: (i, 0)),
            out_shape=shp,
        )(*args)
    in_specs.append(pl.BlockSpec((blk, O), lambda i: (i, 0)))
    args.append(res)
    return pl.pallas_call(
        functools.partial(_edgec_body, k, len(layers), True),
        grid=(B * M // blk,),
        in_specs=in_specs,
        out_specs=[pl.BlockSpec((blk, O), lambda i: (i, 0))] * 2,
        out_shape=[shp, shp],
    )(*args)


def _pool_body(k, g_ref, o_ref):
    acc = g_ref[:, 0, :]
    for j in range(1, k):
        acc = jnp.maximum(acc, g_ref[:, j, :])
    o_ref[...] = acc


def _pool(g3):
    Mq, k, C = g3.shape
    blk = _blk(Mq)
    return pl.pallas_call(
        functools.partial(_pool_body, k),
        grid=(Mq // blk,),
        in_specs=[pl.BlockSpec((blk, k, C), lambda i: (i, 0, 0))],
        out_specs=pl.BlockSpec((blk, C), lambda i: (i, 0)),
        out_shape=jax.ShapeDtypeStruct((Mq, C), F32),
    )(g3)


def _gb(p):
    w, g, b = p
    return w.T, jnp.stack([g, b])


def kernel(x, params):
    B, _, N = x.shape
    xt = jnp.transpose(x, (0, 2, 1))
    node0t = xt[..., :3]
    node1t = node0t[:, ::4, :]
    node2t = node1t[:, ::4, :]
    node3t = node2t[:, ::4, :]
    M1, M2, M3 = N // 4, N // 16, N // 64

    h = _ec(xt, K, params['ec1'])
    x0 = _ec(h.reshape(B, N, 64), K, params['ec2'])
    xt0 = _mm_max(x0.reshape(B, N, 64), *_gb(params['pn3']))

    idxp = _knn(node1t, node0t, K)
    n1f = _pool(_gather(x0, idxp.reshape(-1)).reshape(B * M1, K, 64))

    h = _ec(n1f.reshape(B, M1, 64), K, params['ec4'])
    x1pre, x1 = _ec(h.reshape(B, M1, 64), K, params['ec5'], res=n1f)
    xt1 = _mm_max(x1pre.reshape(B, M1, 64), *_gb(params['pn6']))

    idxp = _knn(node2t, node1t, K)
    n2f = _pool(_gather(x1, idxp.reshape(-1)).reshape(B * M2, K, 64))

    h = _ec(n2f.reshape(B, M2, 64), K, params['ec7'])
    x2pre, x2 = _ec(h.reshape(B, M2, 64), K, params['ec8'], res=n2f)
    xt2 = _mm_max(x2pre.reshape(B, M2, 64), *_gb(params['pn9']))

    idxp = _knn(node3t, node2t, K)
    n3f = _pool(_gather(x2, idxp.reshape(-1)).reshape(B * M3, K, 64))

    h = _ec(n3f.reshape(B, M3, 64), K // 2, params['ec10'])
    x3pre, x3 = _ec(h.reshape(B, M3, 64), K // 2, params['ec11'], res=n3f)
    xt3 = _mm_max(x3pre.reshape(B, M3, 64), *_gb(params['pn12']))

    g = jnp.maximum(jnp.maximum(xt0, xt1), jnp.maximum(xt2, xt3))
    gt = jnp.broadcast_to(g, (B, M3, 1024))
    cat = jnp.concatenate([gt, x3.reshape(B, M3, 64)], axis=-1)
    h = _mm(cat.reshape(B * M3, 1088), *_gb(params['pn13']), act=True)

    idxu = _knn(node2t, node3t, 1)
    hu = _gather(h, idxu.reshape(-1))
    cat = jnp.concatenate([hu, x2], axis=-1)
    h = _mm(cat, *_gb(params['pn14']), act=True)

    idxu = _knn(node1t, node2t, 1)
    hu = _gather(h, idxu.reshape(-1))
    cat = jnp.concatenate([hu, x1], axis=-1)
    h = _mm(cat, *_gb(params['pn15']), act=True)

    idxu = _knn(node0t, node1t, 1)
    hu = _gather(h, idxu.reshape(-1))
    cat = jnp.concatenate([hu, x0], axis=-1)
    h = _mm(cat, *_gb(params['pn16']), act=True)

    out = _mm(h, params['conv17'].T)
    out = jnp.transpose(out.reshape(B, N, 13), (0, 2, 1))
    return (out,
            jnp.transpose(node1t, (0, 2, 1)),
            jnp.transpose(node2t, (0, 2, 1)),
            jnp.transpose(node3t, (0, 2, 1)))


# speedup vs baseline: 8.3536x; 3.0208x over previous
import functools
import jax
import jax.numpy as jnp
from jax.experimental import pallas as pl
from jax.experimental.pallas import tpu as pltpu
from jax.experimental.pallas import tpu_sc as plsc

K = 20
NEG = float('-inf')
F32 = jnp.float32


def _lrelu(x):
    return jnp.where(x >= 0, x, 0.2 * x)


def _blk(m, cap=256):
    b = min(cap, m)
    while m % b:
        b -= 1
    return b


def _knn_body(k, n, q_ref, t_ref, o_ref):
    b = pl.program_id(0)
    q = q_ref[0]
    t = t_ref[0]
    d = 2.0 * jax.lax.dot_general(q, t, (((1,), (1,)), ((), ())),
                                  preferred_element_type=F32)
    d = d - jnp.sum(q * q, axis=1, keepdims=True)
    d = d - jnp.sum(t * t, axis=1)[None, :]
    iota = jax.lax.broadcasted_iota(jnp.int32, d.shape, 1)
    cols = []
    for _ in range(k):
        m = jnp.max(d, axis=-1, keepdims=True)
        sel = jnp.min(jnp.where(d == m, iota, n), axis=-1, keepdims=True)
        cols.append(sel)
        if k > 1:
            d = jnp.where(iota == sel, NEG, d)
    idx = cols[0] if k == 1 else jnp.concatenate(cols, axis=1)
    o_ref[0] = idx + b * n


def _knn(qt, tt, k):
    B, M, C = qt.shape
    N = tt.shape[1]
    blk = _blk(M)
    return pl.pallas_call(
        functools.partial(_knn_body, k, N),
        grid=(B, M // blk),
        in_specs=[
            pl.BlockSpec((1, blk, C), lambda b, i: (b, i, 0)),
            pl.BlockSpec((1, N, C), lambda b, i: (b, 0, 0)),
        ],
        out_specs=pl.BlockSpec((1, blk, k), lambda b, i: (b, i, 0)),
        out_shape=jax.ShapeDtypeStruct((B, M, k), jnp.int32),
    )(qt, tt)


def _gather_body(r, i_ref, t_ref, o_ref):
    idx = i_ref[0, 0]
    iota = jax.lax.broadcasted_iota(jnp.int32, (idx.shape[0], r), 1)
    onehot = (idx[:, None] == iota).astype(jnp.bfloat16)
    t = t_ref[...]
    t1 = t.astype(jnp.bfloat16)
    r1 = t - t1.astype(F32)
    t2 = r1.astype(jnp.bfloat16)
    t3 = (r1 - t2.astype(F32)).astype(jnp.bfloat16)
    acc = jnp.dot(onehot, t1, preferred_element_type=F32)
    acc = acc + jnp.dot(onehot, t2, preferred_element_type=F32)
    acc = acc + jnp.dot(onehot, t3, preferred_element_type=F32)
    o_ref[...] = acc


def _gather_tc(table, idx_flat):
    R, D = table.shape
    Mf = idx_flat.shape[0]
    blk = _blk(Mf)
    idx3 = idx_flat.reshape(Mf // blk, 1, blk)
    return pl.pallas_call(
        functools.partial(_gather_body, R),
        grid=(Mf // blk,),
        in_specs=[
            pl.BlockSpec((1, 1, blk), lambda i: (i, 0, 0)),
            pl.BlockSpec((R, D), lambda i: (0, 0)),
        ],
        out_specs=pl.BlockSpec((blk, D), lambda i: (i, 0)),
        out_shape=jax.ShapeDtypeStruct((Mf, D), F32),
    )(idx3, table)


_SC_NC, _SC_NW, _SC_CH = 2, 32, 128


def _sc_gather_kernel(Mf, D, nc, trips):
    mesh = plsc.VectorSubcoreMesh(core_axis_name="c", subcore_axis_name="s")

    @functools.partial(
        pl.kernel, mesh=mesh,
        out_type=jax.ShapeDtypeStruct((Mf, D), F32),
        scratch_types=[
            pltpu.VMEM((_SC_CH,), jnp.int32),
            pltpu.VMEM((_SC_CH, D), F32),
            pltpu.SemaphoreType.DMA,
        ],
    )
    def k(table_hbm, idx_hbm, out_hbm, idx_v, rows_v, sem):
        wid = jax.lax.axis_index("s") * _SC_NC + jax.lax.axis_index("c")

        def lbody(t, c):
            chunk = t * _SC_NW + wid

            @pl.when(chunk < nc)
            def _():
                base = chunk * _SC_CH
                pltpu.sync_copy(idx_hbm.at[pl.ds(base, _SC_CH)], idx_v)
                pltpu.async_copy(table_hbm.at[idx_v], rows_v, sem).wait()
                pltpu.sync_copy(rows_v, out_hbm.at[pl.ds(base, _SC_CH)])
            return c

        jax.lax.fori_loop(0, trips, lbody, 0)

    return k


def _gather(table, idx_flat):
    Mf = idx_flat.shape[0]
    D = table.shape[1]
    if Mf % _SC_CH:
        return _gather_tc(table, idx_flat)
    Dp = -(-D // 128) * 128
    if Dp != D:
        table = jnp.pad(table, ((0, 0), (0, Dp - D)))
    nc = Mf // _SC_CH
    trips = (nc + _SC_NW - 1) // _SC_NW
    out = _sc_gather_kernel(Mf, Dp, nc, trips)(table, idx_flat)
    return out[:, :D] if Dp != D else out


def _mm_body(act, x_ref, w_ref, gb_ref, o_ref):
    y = jnp.dot(x_ref[...], w_ref[...], preferred_element_type=F32)
    y = y * gb_ref[0][None, :] + gb_ref[1][None, :]
    o_ref[...] = _lrelu(y) if act else y


def _mm_plain_body(x_ref, w_ref, o_ref):
    o_ref[...] = jnp.dot(x_ref[...], w_ref[...], preferred_element_type=F32)


def _mm(xf, wt, gb=None, act=False):
    Mf, Cin = xf.shape
    O = wt.shape[1]
    blk = _blk(Mf)
    if gb is None:
        return pl.pallas_call(
            _mm_plain_body,
            grid=(Mf // blk,),
            in_specs=[
                pl.BlockSpec((blk, Cin), lambda i: (i, 0)),
                pl.BlockSpec((Cin, O), lambda i: (0, 0)),
            ],
            out_specs=pl.BlockSpec((blk, O), lambda i: (i, 0)),
            out_shape=jax.ShapeDtypeStruct((Mf, O), F32),
        )(xf, wt)
    return pl.pallas_call(
        functools.partial(_mm_body, act),
        grid=(Mf // blk,),
        in_specs=[
            pl.BlockSpec((blk, Cin), lambda i: (i, 0)),
            pl.BlockSpec((Cin, O), lambda i: (0, 0)),
            pl.BlockSpec((2, O), lambda i: (0, 0)),
        ],
        out_specs=pl.BlockSpec((blk, O), lambda i: (i, 0)),
        out_shape=jax.ShapeDtypeStruct((Mf, O), F32),
    )(xf, wt, gb)


def _mm_max_body(x_ref, w_ref, gb_ref, o_ref):
    i = pl.program_id(1)
    y = jnp.dot(x_ref[0], w_ref[...], preferred_element_type=F32)
    y = _lrelu(y * gb_ref[0][None, :] + gb_ref[1][None, :])
    blkmax = jnp.max(y, axis=0, keepdims=True)

    @pl.when(i == 0)
    def _():
        o_ref[0] = blkmax

    @pl.when(i > 0)
    def _():
        o_ref[0] = jnp.maximum(o_ref[0], blkmax)


def _mm_max(xt, wt, gb):
    B, M, Cin = xt.shape
    O = wt.shape[1]
    blk = _blk(M)
    return pl.pallas_call(
        _mm_max_body,
        grid=(B, M // blk),
        in_specs=[
            pl.BlockSpec((1, blk, Cin), lambda b, i: (b, i, 0)),
            pl.BlockSpec((Cin, O), lambda b, i: (0, 0)),
            pl.BlockSpec((2, O), lambda b, i: (0, 0)),
        ],
        out_specs=pl.BlockSpec((1, 1, O), lambda b, i: (b, 0, 0)),
        out_shape=jax.ShapeDtypeStruct((B, 1, O), F32),
    )(xt, wt, gb)


def _edgec_body(k, nlayers, has_res, fg_ref, fi_ref, *refs):
    ws = refs[:2 * nlayers]
    rest = refs[2 * nlayers:]
    if has_res:
        res_ref, pre_ref, post_ref = rest
    else:
        (pre_ref,) = rest
    fi = fi_ref[...]
    acc = None
    for j in range(k):
        e = jnp.concatenate([fg_ref[:, j, :] - fi, fi], axis=1)
        h = e
        for li in range(nlayers):
            w_ref, gb_ref = ws[2 * li], ws[2 * li + 1]
            h = jnp.dot(h, w_ref[...], preferred_element_type=F32)
            h = _lrelu(h * gb_ref[0][None, :] + gb_ref[1][None, :])
        acc = h if acc is None else jnp.maximum(acc, h)
    pre_ref[...] = acc
    if has_res:
        post_ref[...] = jnp.maximum(acc + res_ref[...], 0.0)


def _ec(feat, k, layers, res=None):
    B, M, C = feat.shape
    ff = feat.reshape(B * M, C)
    idx = _knn(feat, feat, k)
    fg = _gather(ff, idx.reshape(-1)).reshape(B * M, k, C)
    blk = _blk(B * M)
    in_specs = [
        pl.BlockSpec((blk, k, C), lambda i: (i, 0, 0)),
        pl.BlockSpec((blk, C), lambda i: (i, 0)),
    ]
    args = [fg, ff]
    O = None
    for (w, g, b) in layers:
        o, ci = w.shape
        in_specs.append(pl.BlockSpec((ci, o), lambda i: (0, 0)))
        args.append(w.T)
        in_specs.append(pl.BlockSpec((2, o), lambda i: (0, 0)))
        args.append(jnp.stack([g, b]))
        O = o
    shp = jax.ShapeDtypeStruct((B * M, O), F32)
    if res is None:
        return pl.pallas_call(
            functools.partial(_edgec_body, k, len(layers), False),
            grid=(B * M // blk,),
            in_specs=in_specs,
            out_specs=pl.BlockSpec((blk, O), lambda i: (i, 0)),
            out_shape=shp,
        )(*args)
    in_specs.append(pl.BlockSpec((blk, O), lambda i: (i, 0)))
    args.append(res)
    return pl.pallas_call(
        functools.partial(_edgec_body, k, len(layers), True),
        grid=(B * M // blk,),
        in_specs=in_specs,
        out_specs=[pl.BlockSpec((blk, O), lambda i: (i, 0))] * 2,
        out_shape=[shp, shp],
    )(*args)


def _pool_body(k, g_ref, o_ref):
    acc = g_ref[:, 0, :]
    for j in range(1, k):
        acc = jnp.maximum(acc, g_ref[:, j, :])
    o_ref[...] = acc


def _pool(g3):
    Mq, k, C = g3.shape
    blk = _blk(Mq)
    return pl.pallas_call(
        functools.partial(_pool_body, k),
        grid=(Mq // blk,),
        in_specs=[pl.BlockSpec((blk, k, C), lambda i: (i, 0, 0))],
        out_specs=pl.BlockSpec((blk, C), lambda i: (i, 0)),
        out_shape=jax.ShapeDtypeStruct((Mq, C), F32),
    )(g3)


def _gb(p):
    w, g, b = p
    return w.T, jnp.stack([g, b])


def kernel(x, params):
    B, _, N = x.shape
    xt = jnp.transpose(x, (0, 2, 1))
    node0t = xt[..., :3]
    node1t = node0t[:, ::4, :]
    node2t = node1t[:, ::4, :]
    node3t = node2t[:, ::4, :]
    M1, M2, M3 = N // 4, N // 16, N // 64

    h = _ec(xt, K, params['ec1'])
    x0 = _ec(h.reshape(B, N, 64), K, params['ec2'])
    xt0 = _mm_max(x0.reshape(B, N, 64), *_gb(params['pn3']))

    idxp = _knn(node1t, node0t, K)
    n1f = _pool(_gather(x0, idxp.reshape(-1)).reshape(B * M1, K, 64))

    h = _ec(n1f.reshape(B, M1, 64), K, params['ec4'])
    x1pre, x1 = _ec(h.reshape(B, M1, 64), K, params['ec5'], res=n1f)
    xt1 = _mm_max(x1pre.reshape(B, M1, 64), *_gb(params['pn6']))

    idxp = _knn(node2t, node1t, K)
    n2f = _pool(_gather(x1, idxp.reshape(-1)).reshape(B * M2, K, 64))

    h = _ec(n2f.reshape(B, M2, 64), K, params['ec7'])
    x2pre, x2 = _ec(h.reshape(B, M2, 64), K, params['ec8'], res=n2f)
    xt2 = _mm_max(x2pre.reshape(B, M2, 64), *_gb(params['pn9']))

    idxp = _knn(node3t, node2t, K)
    n3f = _pool(_gather(x2, idxp.reshape(-1)).reshape(B * M3, K, 64))

    h = _ec(n3f.reshape(B, M3, 64), K // 2, params['ec10'])
    x3pre, x3 = _ec(h.reshape(B, M3, 64), K // 2, params['ec11'], res=n3f)
    xt3 = _mm_max(x3pre.reshape(B, M3, 64), *_gb(params['pn12']))

    g = jnp.maximum(jnp.maximum(xt0, xt1), jnp.maximum(xt2, xt3))
    gt = jnp.broadcast_to(g, (B, M3, 1024))
    cat = jnp.concatenate([gt, x3.reshape(B, M3, 64)], axis=-1)
    h = _mm(cat.reshape(B * M3, 1088), *_gb(params['pn13']), act=True)

    idxu = _knn(node2t, node3t, 1)
    hu = _gather(h, idxu.reshape(-1))
    cat = jnp.concatenate([hu, x2], axis=-1)
    h = _mm(cat, *_gb(params['pn14']), act=True)

    idxu = _knn(node1t, node2t, 1)
    hu = _gather(h, idxu.reshape(-1))
    cat = jnp.concatenate([hu, x1], axis=-1)
    h = _mm(cat, *_gb(params['pn15']), act=True)

    idxu = _knn(node0t, node1t, 1)
    hu = _gather(h, idxu.reshape(-1))
    cat = jnp.concatenate([hu, x0], axis=-1)
    h = _mm(cat, *_gb(params['pn16']), act=True)

    out = _mm(h, params['conv17'].T)
    out = jnp.transpose(out.reshape(B, N, 13), (0, 2, 1))
    return (out,
            jnp.transpose(node1t, (0, 2, 1)),
            jnp.transpose(node2t, (0, 2, 1)),
            jnp.transpose(node3t, (0, 2, 1)))
